# baseline (speedup 1.0000x reference)
import functools

import jax
import jax.numpy as jnp
from jax import lax
from jax.experimental import pallas as pl
from jax.experimental.pallas import tpu as pltpu
from jax.experimental.pallas import tpu_sc as plsc

NC, NS, L = 2, 16, 16
NW = NC * NS
IC = 128


def _shade_fn(n_pix, n_c, w_img, blk, p2f_hbm, bary_hbm, vf_hbm, faces_hbm,
              out_hbm, p2f_v, idx_v, bary_v, fverts_v, vidx_v, rows_v, out_t,
              sem):
    n_per_w = n_pix // NW
    n_blocks = n_per_w // blk
    rows_blk = blk // w_img
    chunks_row = w_img // L
    wid = lax.axis_index("s") * NC + lax.axis_index("c")
    base0 = wid * n_per_w

    def do_block(b, _):
        base = base0 + b * blk
        rowbase = base // w_img
        pltpu.sync_copy(p2f_hbm.at[pl.ds(base, blk)], p2f_v)
        pltpu.sync_copy(bary_hbm.at[pl.ds(rowbase, rows_blk)], bary_v)

        def clamp_body(i, _):
            f = p2f_v[pl.ds(i * L, L)]
            idx_v[i // (IC // L), pl.ds((i % (IC // L)) * L, L)] = (
                jnp.maximum(f, 0) >> 1)
            return _

        lax.fori_loop(0, blk // L, clamp_body, None, unroll=4)

        cps = [
            pltpu.async_copy(faces_hbm.at[idx_v.at[j]],
                             fverts_v.at[pl.ds(j * IC, IC)], sem)
            for j in range(blk // IC)
        ]
        for cp in cps:
            cp.wait()

        def flat_body(q, _):
            t = 16 * q + lax.iota(jnp.int32, L)
            p = t // 3
            fcl = jnp.maximum(plsc.load_gather(p2f_v, [p]), 0)
            col = (fcl & 1) * 8 + t % 3
            val = plsc.load_gather(fverts_v, [p, col])
            vidx_v[q // (IC // L), pl.ds((q % (IC // L)) * L, L)] = val
            return _

        lax.fori_loop(0, 3 * blk // L, flat_body, None, unroll=4)

        cps = [
            pltpu.async_copy(vf_hbm.at[vidx_v.at[m]],
                             rows_v.at[pl.ds(m * IC, IC)], sem)
            for m in range(3 * blk // IC)
        ]
        for cp in cps:
            cp.wait()

        def px_body(i, _):
            row_l = i // chunks_row
            w0 = (i % chunks_row) * L
            l = i * L + lax.iota(jnp.int32, L)
            f = p2f_v[pl.ds(i * L, L)]
            mf = jnp.where(f >= 0, jnp.float32(1.0), jnp.float32(0.0))
            w = [bary_v[row_l, j, 0, pl.ds(w0, L)] * mf for j in range(3)]
            r = [3 * l + j for j in range(3)]
            for c in range(n_c):
                cc = jnp.full((L,), c, jnp.int32)
                acc = w[0] * plsc.load_gather(rows_v, [r[0], cc])
                acc += w[1] * plsc.load_gather(rows_v, [r[1], cc])
                acc += w[2] * plsc.load_gather(rows_v, [r[2], cc])
                out_t[row_l, c, pl.ds(w0, L)] = acc
            return _

        lax.fori_loop(0, blk // L, px_body, None, unroll=2)
        pltpu.sync_copy(out_t, out_hbm.at[pl.ds(rowbase, rows_blk)])
        return _

    lax.fori_loop(0, n_blocks, do_block, None)


def _shade(p2f, bary_t, vf, faces16, n_pix, w_img):
    n_c = vf.shape[1]
    blk = 3 * w_img
    mesh = plsc.VectorSubcoreMesh(core_axis_name="c", subcore_axis_name="s",
                                  num_cores=NC, num_subcores=NS)
    return pl.kernel(
        functools.partial(_shade_fn, n_pix, n_c, w_img, blk),
        out_type=jax.ShapeDtypeStruct((n_pix // w_img, n_c, w_img),
                                      jnp.float32),
        mesh=mesh,
        compiler_params=pltpu.CompilerParams(needs_layout_passes=False,
                                             use_tc_tiling_on_sc=False),
        scratch_types=[
            pltpu.VMEM((blk,), jnp.int32),
            pltpu.VMEM((blk // IC, IC), jnp.int32),
            pltpu.VMEM((blk // w_img, 3, 2, w_img), jnp.float32),
            pltpu.VMEM((blk, 16), jnp.int32),
            pltpu.VMEM((3 * blk // IC, IC), jnp.int32),
            pltpu.VMEM((3 * blk, n_c), jnp.float32),
            pltpu.VMEM((blk // w_img, n_c, w_img), jnp.float32),
            pltpu.SemaphoreType.DMA,
        ],
    )(p2f, bary_t, vf, faces16)


def kernel(pix_to_face, bary_coords, verts_features, faces):
    b, h, w, k = pix_to_face.shape
    v, c = verts_features.shape
    n = b * h * w
    p2f = pix_to_face[..., 0].reshape(n).astype(jnp.int32)
    bary_t = bary_coords.transpose(0, 1, 4, 3, 2).reshape(b * h, 3, k, w)
    faces16 = jnp.concatenate(
        [faces.astype(jnp.int32),
         jnp.zeros((faces.shape[0], 5), jnp.int32)],
        axis=1).reshape(faces.shape[0] // 2, 16)
    out = _shade(p2f, bary_t, verts_features, faces16, n, w)
    return out.reshape(b, h, c, w).transpose(0, 1, 3, 2)


# speedup vs baseline: 1.0831x; 1.0831x over previous
import functools

import jax
import jax.numpy as jnp
from jax import lax
from jax.experimental import pallas as pl
from jax.experimental.pallas import tpu as pltpu
from jax.experimental.pallas import tpu_sc as plsc

NC, NS, L = 2, 16, 16
NW = NC * NS
IC = 128


def _shade_fn(n_pix, n_c, w_img, blk, p2f_hbm, bary_hbm, vf_hbm, faces_hbm,
              out_hbm, p2f_v, idx_v, bary_v, fverts_v, vidx_v, rows_v, out_t,
              fs0, fs1, fs2, vs0, vs1, vs2):
    fsems = [fs0, fs1, fs2]
    vsems = [vs0, vs1, vs2]
    n_per_w = n_pix // NW
    n_blocks = n_per_w // blk
    rows_blk = blk // w_img
    chunks_row = w_img // L
    wid = lax.axis_index("s") * NC + lax.axis_index("c")
    base0 = wid * n_per_w

    def do_block(b, _):
        base = base0 + b * blk
        rowbase = base // w_img
        pltpu.sync_copy(p2f_hbm.at[pl.ds(base, blk)], p2f_v)
        pltpu.sync_copy(bary_hbm.at[pl.ds(rowbase, rows_blk)], bary_v)

        def clamp_body(i, _):
            f = p2f_v[pl.ds(i * L, L)]
            idx_v[i // (IC // L), pl.ds((i % (IC // L)) * L, L)] = (
                jnp.maximum(f, 0))
            return _

        lax.fori_loop(0, blk // L, clamp_body, None, unroll=4)

        ng = rows_blk
        jg = blk // IC // ng
        mg = 3 * blk // IC // ng
        qg = 3 * blk // L // ng
        ig = blk // L // ng

        def flat_chunk(q, _):
            t = 16 * q + lax.iota(jnp.int32, L)
            val = plsc.load_gather(fverts_v, [t // 3, t % 3])
            vidx_v[q // (IC // L), pl.ds((q % (IC // L)) * L, L)] = val
            return _

        def px_chunk(i, _):
            row_l = i // chunks_row
            w0 = (i % chunks_row) * L
            l = i * L + lax.iota(jnp.int32, L)
            f = p2f_v[pl.ds(i * L, L)]
            mf = jnp.where(f >= 0, jnp.float32(1.0), jnp.float32(0.0))
            w = [bary_v[row_l, j, 0, pl.ds(w0, L)] * mf for j in range(3)]
            r = [3 * l + j for j in range(3)]
            for c in range(n_c):
                cc = jnp.full((L,), c, jnp.int32)
                acc = w[0] * plsc.load_gather(rows_v, [r[0], cc])
                acc += w[1] * plsc.load_gather(rows_v, [r[1], cc])
                acc += w[2] * plsc.load_gather(rows_v, [r[2], cc])
                out_t[row_l, c, pl.ds(w0, L)] = acc
            return _

        fcp = [[pltpu.async_copy(faces_hbm.at[idx_v.at[g * jg + j]],
                                 fverts_v.at[pl.ds((g * jg + j) * IC, IC)],
                                 fsems[g])
                for j in range(jg)] for g in range(ng)]
        vcp = []
        for g in range(ng):
            for cp in fcp[g]:
                cp.wait()
            lax.fori_loop(g * qg, (g + 1) * qg, flat_chunk, None, unroll=4)
            vcp.append([pltpu.async_copy(vf_hbm.at[vidx_v.at[g * mg + m]],
                                         rows_v.at[pl.ds((g * mg + m) * IC,
                                                         IC)],
                                         vsems[g])
                        for m in range(mg)])
        for g in range(ng):
            for cp in vcp[g]:
                cp.wait()
            lax.fori_loop(g * ig, (g + 1) * ig, px_chunk, None, unroll=2)
        pltpu.sync_copy(out_t, out_hbm.at[pl.ds(rowbase, rows_blk)])
        return _

    lax.fori_loop(0, n_blocks, do_block, None)


def _shade(p2f, bary_t, vf, faces16, n_pix, w_img):
    n_c = vf.shape[1]
    blk = 3 * w_img
    mesh = plsc.VectorSubcoreMesh(core_axis_name="c", subcore_axis_name="s",
                                  num_cores=NC, num_subcores=NS)
    return pl.kernel(
        functools.partial(_shade_fn, n_pix, n_c, w_img, blk),
        out_type=jax.ShapeDtypeStruct((n_pix // w_img, n_c, w_img),
                                      jnp.float32),
        mesh=mesh,
        compiler_params=pltpu.CompilerParams(needs_layout_passes=False,
                                             use_tc_tiling_on_sc=False),
        scratch_types=[
            pltpu.VMEM((blk,), jnp.int32),
            pltpu.VMEM((blk // IC, IC), jnp.int32),
            pltpu.VMEM((blk // w_img, 3, 2, w_img), jnp.float32),
            pltpu.VMEM((blk, 16), jnp.int32),
            pltpu.VMEM((3 * blk // IC, IC), jnp.int32),
            pltpu.VMEM((3 * blk, n_c), jnp.float32),
            pltpu.VMEM((blk // w_img, n_c, w_img), jnp.float32),
            pltpu.SemaphoreType.DMA,
            pltpu.SemaphoreType.DMA,
            pltpu.SemaphoreType.DMA,
            pltpu.SemaphoreType.DMA,
            pltpu.SemaphoreType.DMA,
            pltpu.SemaphoreType.DMA,
        ],
    )(p2f, bary_t, vf, faces16)


def kernel(pix_to_face, bary_coords, verts_features, faces):
    b, h, w, k = pix_to_face.shape
    v, c = verts_features.shape
    n = b * h * w
    p2f = pix_to_face[..., 0].reshape(n).astype(jnp.int32)
    bary_t = bary_coords.transpose(0, 1, 4, 3, 2).reshape(b * h, 3, k, w)
    faces16 = jnp.concatenate(
        [faces.astype(jnp.int32),
         jnp.zeros((faces.shape[0], 13), jnp.int32)], axis=1)
    out = _shade(p2f, bary_t, verts_features, faces16, n, w)
    return out.reshape(b, h, c, w).transpose(0, 1, 3, 2)
